# baseline (speedup 1.0000x reference)
import functools
import math

import jax
import jax.numpy as jnp
import numpy as np
from jax.experimental import pallas as pl
from jax.experimental.pallas import tpu as pltpu

B, T, E, H = 4, 2048, 1024, 16
DH = E // H
TOPK = int(3 * math.log(T))
NF = T // 2 + 1
NFP = 1152
FB = 384
EB = 512
TB = 256
NG = B
GW = H // NG * DH

_PERM = [h for c in range(NG) for h in range(c, H, NG)]


@functools.lru_cache(maxsize=1)
def _dft_consts():
    f = np.arange(NFP, dtype=np.int64)[:, None]
    t = np.arange(T, dtype=np.int64)[None, :]
    ang = 2.0 * np.pi * ((f * t) % T).astype(np.float64) / T
    valid = (f < NF).astype(np.float64)
    fc = (np.cos(ang) * valid).astype(np.float32)
    fs = (-np.sin(ang) * valid).astype(np.float32)
    wgt = np.where((f == 0) | (f == T // 2), 1.0, 2.0) * valid / (T * E)
    ict = (np.cos(ang) * wgt).astype(np.float32)
    ist = (-np.sin(ang) * wgt).astype(np.float32)
    return fc, fs, ict, ist


def _projspec_body(hs_ref, w_ref, bias_ref, fc_ref, fs_ref,
                   v_ref, sr_ref, si_ref, cs, ss):
    t = pl.program_id(1)
    dot = functools.partial(jnp.dot, preferred_element_type=jnp.float32)
    acc = jax.lax.dot_general(
        hs_ref[0], w_ref[...],
        (((1,), (1,)), ((), ())),
        preferred_element_type=jnp.float32) + bias_ref[0:1, :]
    v_ref[0] = acc[:, 2 * E:]
    qk = acc[:, 0:2 * E]
    CH = 512
    for j in range(0, 2 * E, CH):
        dc = dot(fc_ref[...], qk[:, j:j + CH])
        ds_ = dot(fs_ref[...], qk[:, j:j + CH])

        @pl.when(t == 0)
        def _():
            cs[:, j:j + CH] = dc
            ss[:, j:j + CH] = ds_

        @pl.when(t != 0)
        def _():
            cs[:, j:j + CH] += dc
            ss[:, j:j + CH] += ds_

    @pl.when(t == T // TB - 1)
    def _():
        sr = jnp.zeros((NFP, 1), jnp.float32)
        si = jnp.zeros((NFP, 1), jnp.float32)
        for j in range(0, E, CH):
            qc, kc = cs[:, j:j + CH], cs[:, E + j:E + j + CH]
            qs, ks = ss[:, j:j + CH], ss[:, E + j:E + j + CH]
            sr += jnp.sum(qc * kc + qs * ks, axis=1, keepdims=True)
            si += jnp.sum(qs * kc - qc * ks, axis=1, keepdims=True)
        sr_ref[0] = sr
        si_ref[0] = si


def _invtopk_body(srt_ref, sit_ref, ict_ref, ist_ref, d_ref, w_ref):
    dot = functools.partial(jnp.dot, preferred_element_type=jnp.float32)
    vals = dot(srt_ref[...], ict_ref[...]) + dot(sit_ref[...], ist_ref[...])
    lane = jax.lax.broadcasted_iota(jnp.int32, (B, T), 1)
    out_lane = jax.lax.broadcasted_iota(jnp.int32, (B, 128), 1)
    dacc = jnp.zeros((B, 128), jnp.int32)
    wacc = jnp.zeros((B, 128), jnp.float32)
    for i in range(TOPK):
        cur = jnp.max(vals, axis=1, keepdims=True)
        idx = jnp.min(jnp.where(vals == cur, lane, T),
                      axis=1, keepdims=True)
        dacc = jnp.where(out_lane == i, idx, dacc)
        wacc = jnp.where(out_lane == i, cur, wacc)
        vals = jnp.where(lane == idx, -jnp.inf, vals)
    d_ref[...] = dacc
    w_ref[...] = wacc


def _agg_body(d_ref, w_ref, v_ref, ow_ref, ob_ref, out_ref, vd0, vd1):
    c = pl.program_id(1)
    vd0[0:T] = v_ref[0, :, 0:128]
    vd0[T:2 * T] = v_ref[0, :, 0:128]
    vd1[0:T] = v_ref[0, :, 128:256]
    vd1[T:2 * T] = v_ref[0, :, 128:256]
    acc0 = jnp.zeros((T, 128), jnp.float32)
    acc1 = jnp.zeros((T, 128), jnp.float32)
    for i in range(TOPK):
        d = d_ref[c, i]
        w = w_ref[c, i]
        acc0 = acc0 + w * vd0[pl.ds(d, T), :]
        acc1 = acc1 + w * vd1[pl.ds(d, T), :]
    acc = jnp.concatenate([acc0, acc1], axis=1)
    ctr = jax.lax.dot_general(
        acc, ow_ref[...],
        (((1,), (1,)), ((), ())),
        preferred_element_type=jnp.float32)

    @pl.when(c == 0)
    def _():
        out_ref[0] = jnp.broadcast_to(ob_ref[0:1, :], (T, E))

    out_ref[0] += ctr


def kernel(hidden_states, q_w, q_b, k_w, k_b, v_w, v_b, o_w, o_b):
    perm = jnp.asarray(_PERM, dtype=jnp.int32)
    v_w_p = v_w.reshape(H, DH, E)[perm].reshape(E, E)
    v_b_p = v_b.reshape(H, DH)[perm].reshape(E)
    o_w_p = o_w.reshape(E, H, DH)[:, perm].reshape(E, E)
    w_all = jnp.concatenate([q_w, k_w, v_w_p], axis=0)
    bias_all = jnp.tile(
        jnp.concatenate([q_b, k_b, v_b_p])[None, :], (8, 1))

    fc, fs, ict, ist = (jnp.asarray(a) for a in _dft_consts())
    v, sr3, si3 = pl.pallas_call(
        _projspec_body,
        grid=(B, T // TB),
        in_specs=[
            pl.BlockSpec((1, TB, E), lambda b, t: (b, t, 0)),
            pl.BlockSpec((3 * E, E), lambda b, t: (0, 0)),
            pl.BlockSpec((8, 3 * E), lambda b, t: (0, 0)),
            pl.BlockSpec((NFP, TB), lambda b, t: (0, t)),
            pl.BlockSpec((NFP, TB), lambda b, t: (0, t)),
        ],
        out_specs=[
            pl.BlockSpec((1, TB, E), lambda b, t: (b, t, 0)),
            pl.BlockSpec((1, NFP, 1), lambda b, t: (b, 0, 0)),
            pl.BlockSpec((1, NFP, 1), lambda b, t: (b, 0, 0)),
        ],
        out_shape=[
            jax.ShapeDtypeStruct((B, T, E), jnp.float32),
            jax.ShapeDtypeStruct((B, NFP, 1), jnp.float32),
            jax.ShapeDtypeStruct((B, NFP, 1), jnp.float32),
        ],
        scratch_shapes=[
            pltpu.VMEM((NFP, 2 * E), jnp.float32),
            pltpu.VMEM((NFP, 2 * E), jnp.float32),
        ],
    )(hidden_states, w_all, bias_all, fc, fs)

    delays, weights = pl.pallas_call(
        _invtopk_body,
        in_specs=[
            pl.BlockSpec((B, NFP), lambda: (0, 0)),
            pl.BlockSpec((B, NFP), lambda: (0, 0)),
            pl.BlockSpec((NFP, T), lambda: (0, 0)),
            pl.BlockSpec((NFP, T), lambda: (0, 0)),
        ],
        out_specs=[
            pl.BlockSpec((B, 128), lambda: (0, 0)),
            pl.BlockSpec((B, 128), lambda: (0, 0)),
        ],
        out_shape=[
            jax.ShapeDtypeStruct((B, 128), jnp.int32),
            jax.ShapeDtypeStruct((B, 128), jnp.float32),
        ],
    )(sr3.reshape(B, NFP), si3.reshape(B, NFP), ict, ist)

    ob = jnp.tile(o_b[None, :], (8, 1))
    out = pl.pallas_call(
        _agg_body,
        grid_spec=pltpu.PrefetchScalarGridSpec(
            num_scalar_prefetch=2,
            grid=(B, NG),
            in_specs=[
                pl.BlockSpec((1, T, GW), lambda b, c, dd, ww: (b, 0, c)),
                pl.BlockSpec((E, GW), lambda b, c, dd, ww: (0, c)),
                pl.BlockSpec((8, E), lambda b, c, dd, ww: (0, 0)),
            ],
            out_specs=pl.BlockSpec((1, T, E), lambda b, c, dd, ww: (b, 0, 0)),
            scratch_shapes=[pltpu.VMEM((2 * T, 128), jnp.float32),
                            pltpu.VMEM((2 * T, 128), jnp.float32)],
        ),
        out_shape=jax.ShapeDtypeStruct((B, T, E), jnp.float32),
    )(delays, weights, v, o_w_p, ob)
    return out


# speedup vs baseline: 1.1704x; 1.1704x over previous
import functools
import math

import jax
import jax.numpy as jnp
import numpy as np
from jax.experimental import pallas as pl
from jax.experimental.pallas import tpu as pltpu

B, T, E, H = 4, 2048, 1024, 16
DH = E // H
TOPK = int(3 * math.log(T))
NF = T // 2 + 1
NFP = 1152
FB = 384
EB = 512
TB = 256
NG = B
GW = H // NG * DH

_PERM = [h for c in range(NG) for h in range(c, H, NG)]


@functools.lru_cache(maxsize=1)
def _dft_consts():
    f = np.arange(NFP, dtype=np.int64)[:, None]
    t = np.arange(T, dtype=np.int64)[None, :]
    ang = 2.0 * np.pi * ((f * t) % T).astype(np.float64) / T
    valid = (f < NF).astype(np.float64)
    fc = (np.cos(ang) * valid).astype(np.float32)
    fs = (-np.sin(ang) * valid).astype(np.float32)
    wgt = np.where((f == 0) | (f == T // 2), 1.0, 2.0) * valid / (T * E)
    ict = (np.cos(ang) * wgt).astype(np.float32)
    ist = (-np.sin(ang) * wgt).astype(np.float32)
    return fc, fs, ict, ist


def _qkv_body(hs_ref, w_ref, bias_ref, out_ref):
    acc = jax.lax.dot_general(
        hs_ref[0], w_ref[...],
        (((1,), (1,)), ((), ())),
        preferred_element_type=jnp.float32)
    out_ref[0] = acc + bias_ref[0:1, :]


def _corr_body(q_ref, k_ref, fc_ref, fs_ref, sr_ref, si_ref):
    e = pl.program_id(2)
    q = q_ref[0]
    k = k_ref[0]
    dot = functools.partial(jnp.dot, preferred_element_type=jnp.float32)
    qc = dot(fc_ref[...], q)
    qs = dot(fs_ref[...], q)
    kc = dot(fc_ref[...], k)
    ks = dot(fs_ref[...], k)
    sr = jnp.sum(qc * kc + qs * ks, axis=1, keepdims=True)
    si = jnp.sum(qs * kc - qc * ks, axis=1, keepdims=True)

    @pl.when(e == 0)
    def _():
        sr_ref[0] = jnp.zeros_like(sr_ref[0])
        si_ref[0] = jnp.zeros_like(si_ref[0])

    sr_ref[0] += sr
    si_ref[0] += si


def _invtopk_body(srt_ref, sit_ref, ict_ref, ist_ref, d_ref, w_ref):
    dot = functools.partial(jnp.dot, preferred_element_type=jnp.float32)
    vals = dot(srt_ref[...], ict_ref[...]) + dot(sit_ref[...], ist_ref[...])
    lane = jax.lax.broadcasted_iota(jnp.int32, (B, T), 1)
    out_lane = jax.lax.broadcasted_iota(jnp.int32, (B, 128), 1)
    dacc = jnp.zeros((B, 128), jnp.int32)
    wacc = jnp.zeros((B, 128), jnp.float32)
    for i in range(TOPK):
        cur = jnp.max(vals, axis=1, keepdims=True)
        idx = jnp.min(jnp.where(vals == cur, lane, T),
                      axis=1, keepdims=True)
        dacc = jnp.where(out_lane == i, idx, dacc)
        wacc = jnp.where(out_lane == i, cur, wacc)
        vals = jnp.where(lane == idx, -jnp.inf, vals)
    d_ref[...] = dacc
    w_ref[...] = wacc


def _agg_body(d_ref, w_ref, v_ref, ow_ref, ob_ref, out_ref, vd0, vd1):
    c = pl.program_id(1)
    vd0[0:T] = v_ref[0, :, 0:128]
    vd0[T:2 * T] = v_ref[0, :, 0:128]
    vd1[0:T] = v_ref[0, :, 128:256]
    vd1[T:2 * T] = v_ref[0, :, 128:256]
    acc0 = jnp.zeros((T, 128), jnp.float32)
    acc1 = jnp.zeros((T, 128), jnp.float32)
    for i in range(TOPK):
        d = d_ref[c, i]
        w = w_ref[c, i]
        acc0 = acc0 + w * vd0[pl.ds(d, T), :]
        acc1 = acc1 + w * vd1[pl.ds(d, T), :]
    acc = jnp.concatenate([acc0, acc1], axis=1)
    ctr = jax.lax.dot_general(
        acc, ow_ref[...],
        (((1,), (1,)), ((), ())),
        preferred_element_type=jnp.float32)

    @pl.when(c == 0)
    def _():
        out_ref[0] = jnp.broadcast_to(ob_ref[0:1, :], (T, E))

    out_ref[0] += ctr


def kernel(hidden_states, q_w, q_b, k_w, k_b, v_w, v_b, o_w, o_b):
    perm = jnp.asarray(_PERM, dtype=jnp.int32)
    v_w_p = v_w.reshape(H, DH, E)[perm].reshape(E, E)
    v_b_p = v_b.reshape(H, DH)[perm].reshape(E)
    o_w_p = o_w.reshape(E, H, DH)[:, perm].reshape(E, E)
    w_all = jnp.concatenate([q_w, k_w, v_w_p], axis=0)
    bias_all = jnp.tile(
        jnp.concatenate([q_b, k_b, v_b_p])[None, :], (8, 1))

    qkv = pl.pallas_call(
        _qkv_body,
        grid=(B, T // TB),
        in_specs=[
            pl.BlockSpec((1, TB, E), lambda b, t: (b, t, 0)),
            pl.BlockSpec((3 * E, E), lambda b, t: (0, 0)),
            pl.BlockSpec((8, 3 * E), lambda b, t: (0, 0)),
        ],
        out_specs=pl.BlockSpec((1, TB, 3 * E), lambda b, t: (b, t, 0)),
        out_shape=jax.ShapeDtypeStruct((B, T, 3 * E), jnp.float32),
    )(hidden_states, w_all, bias_all)

    fc, fs, ict, ist = (jnp.asarray(a) for a in _dft_consts())
    ne = E // EB
    sr3, si3 = pl.pallas_call(
        _corr_body,
        grid=(NFP // FB, B, ne),
        in_specs=[
            pl.BlockSpec((1, T, EB), lambda f, b, e: (b, 0, e)),
            pl.BlockSpec((1, T, EB), lambda f, b, e: (b, 0, ne + e)),
            pl.BlockSpec((FB, T), lambda f, b, e: (f, 0)),
            pl.BlockSpec((FB, T), lambda f, b, e: (f, 0)),
        ],
        out_specs=[
            pl.BlockSpec((1, FB, 1), lambda f, b, e: (b, f, 0)),
            pl.BlockSpec((1, FB, 1), lambda f, b, e: (b, f, 0)),
        ],
        out_shape=[
            jax.ShapeDtypeStruct((B, NFP, 1), jnp.float32),
            jax.ShapeDtypeStruct((B, NFP, 1), jnp.float32),
        ],
    )(qkv, qkv, fc, fs)

    delays, weights = pl.pallas_call(
        _invtopk_body,
        in_specs=[
            pl.BlockSpec((B, NFP), lambda: (0, 0)),
            pl.BlockSpec((B, NFP), lambda: (0, 0)),
            pl.BlockSpec((NFP, T), lambda: (0, 0)),
            pl.BlockSpec((NFP, T), lambda: (0, 0)),
        ],
        out_specs=[
            pl.BlockSpec((B, 128), lambda: (0, 0)),
            pl.BlockSpec((B, 128), lambda: (0, 0)),
        ],
        out_shape=[
            jax.ShapeDtypeStruct((B, 128), jnp.int32),
            jax.ShapeDtypeStruct((B, 128), jnp.float32),
        ],
    )(sr3.reshape(B, NFP), si3.reshape(B, NFP), ict, ist)

    ob = jnp.tile(o_b[None, :], (8, 1))
    out = pl.pallas_call(
        _agg_body,
        grid_spec=pltpu.PrefetchScalarGridSpec(
            num_scalar_prefetch=2,
            grid=(B, NG),
            in_specs=[
                pl.BlockSpec((1, T, GW),
                             lambda b, c, dd, ww: (b, 0, 2 * E // GW + c)),
                pl.BlockSpec((E, GW), lambda b, c, dd, ww: (0, c)),
                pl.BlockSpec((8, E), lambda b, c, dd, ww: (0, 0)),
            ],
            out_specs=pl.BlockSpec((1, T, E), lambda b, c, dd, ww: (b, 0, 0)),
            scratch_shapes=[pltpu.VMEM((2 * T, 128), jnp.float32),
                            pltpu.VMEM((2 * T, 128), jnp.float32)],
        ),
        out_shape=jax.ShapeDtypeStruct((B, T, E), jnp.float32),
    )(delays, weights, qkv, o_w_p, ob)
    return out


# speedup vs baseline: 1.1828x; 1.0106x over previous
import functools
import math

import jax
import jax.numpy as jnp
import numpy as np
from jax.experimental import pallas as pl
from jax.experimental.pallas import tpu as pltpu

B, T, E, H = 4, 2048, 1024, 16
DH = E // H
TOPK = int(3 * math.log(T))
NF = T // 2 + 1
NFP = 1152
FB = 384
EB = 512
TB = 256
NG = B
GW = H // NG * DH

_PERM = [h for c in range(NG) for h in range(c, H, NG)]


@functools.lru_cache(maxsize=1)
def _dft_consts():
    f = np.arange(NFP, dtype=np.int64)[:, None]
    t = np.arange(T, dtype=np.int64)[None, :]
    ang = 2.0 * np.pi * ((f * t) % T).astype(np.float64) / T
    valid = (f < NF).astype(np.float64)
    fc = (np.cos(ang) * valid).astype(np.float32)
    fs = (-np.sin(ang) * valid).astype(np.float32)
    wgt = np.where((f == 0) | (f == T // 2), 1.0, 2.0) * valid / (T * E)
    ict = (np.cos(ang) * wgt).astype(np.float32)
    ist = (-np.sin(ang) * wgt).astype(np.float32)
    return fc, fs, ict, ist


def _qkv_body(hs_ref, w_ref, bias_ref, out_ref):
    acc = jax.lax.dot_general(
        hs_ref[0], w_ref[...],
        (((1,), (1,)), ((), ())),
        preferred_element_type=jnp.float32)
    out_ref[0] = acc + bias_ref[0:1, :]


def _corr_body(q_ref, k_ref, fc_ref, fs_ref, sr_ref, si_ref):
    e = pl.program_id(1)
    q = q_ref[0]
    k = k_ref[0]
    dot = functools.partial(jnp.dot, preferred_element_type=jnp.float32)
    qc = dot(fc_ref[...], q)
    qs = dot(fs_ref[...], q)
    kc = dot(fc_ref[...], k)
    ks = dot(fs_ref[...], k)
    sr = jnp.sum(qc * kc + qs * ks, axis=1, keepdims=True)
    si = jnp.sum(qs * kc - qc * ks, axis=1, keepdims=True)

    @pl.when(e == 0)
    def _():
        sr_ref[0] = jnp.zeros_like(sr_ref[0])
        si_ref[0] = jnp.zeros_like(si_ref[0])

    sr_ref[0] += sr
    si_ref[0] += si


def _invtopk_body(srt_ref, sit_ref, ict_ref, ist_ref, d_ref, w_ref):
    dot = functools.partial(jnp.dot, preferred_element_type=jnp.float32)
    vals = dot(srt_ref[...], ict_ref[...]) + dot(sit_ref[...], ist_ref[...])
    lane = jax.lax.broadcasted_iota(jnp.int32, (B, T), 1)
    out_lane = jax.lax.broadcasted_iota(jnp.int32, (B, 128), 1)
    dacc = jnp.zeros((B, 128), jnp.int32)
    wacc = jnp.zeros((B, 128), jnp.float32)
    for i in range(TOPK):
        cur = jnp.max(vals, axis=1, keepdims=True)
        idx = jnp.min(jnp.where(vals == cur, lane, T),
                      axis=1, keepdims=True)
        dacc = jnp.where(out_lane == i, idx, dacc)
        wacc = jnp.where(out_lane == i, cur, wacc)
        vals = jnp.where(lane == idx, -jnp.inf, vals)
    d_ref[...] = dacc
    w_ref[...] = wacc


def _agg_body(d_ref, w_ref, v_ref, ow_ref, ob_ref, out_ref, vd0, vd1):
    c = pl.program_id(1)
    vd0[0:T] = v_ref[0, :, 0:128]
    vd0[T:2 * T] = v_ref[0, :, 0:128]
    vd1[0:T] = v_ref[0, :, 128:256]
    vd1[T:2 * T] = v_ref[0, :, 128:256]
    acc0 = jnp.zeros((T, 128), jnp.float32)
    acc1 = jnp.zeros((T, 128), jnp.float32)
    for i in range(TOPK):
        d = d_ref[c, i]
        w = w_ref[c, i]
        acc0 = acc0 + w * vd0[pl.ds(d, T), :]
        acc1 = acc1 + w * vd1[pl.ds(d, T), :]
    acc = jnp.concatenate([acc0, acc1], axis=1)
    ctr = jax.lax.dot_general(
        acc, ow_ref[...],
        (((1,), (1,)), ((), ())),
        preferred_element_type=jnp.float32)

    @pl.when(c == 0)
    def _():
        out_ref[0] = jnp.broadcast_to(ob_ref[0:1, :], (T, E))

    out_ref[0] += ctr


def kernel(hidden_states, q_w, q_b, k_w, k_b, v_w, v_b, o_w, o_b):
    perm = jnp.asarray(_PERM, dtype=jnp.int32)
    v_w_p = v_w.reshape(H, DH, E)[perm].reshape(E, E)
    v_b_p = v_b.reshape(H, DH)[perm].reshape(E)
    o_w_p = o_w.reshape(E, H, DH)[:, perm].reshape(E, E)
    w_all = jnp.concatenate([q_w, k_w, v_w_p], axis=0)
    bias_all = jnp.tile(
        jnp.concatenate([q_b, k_b, v_b_p])[None, :], (8, 1))

    qkv = pl.pallas_call(
        _qkv_body,
        grid=(B, T // TB),
        in_specs=[
            pl.BlockSpec((1, TB, E), lambda b, t: (b, t, 0)),
            pl.BlockSpec((3 * E, E), lambda b, t: (0, 0)),
            pl.BlockSpec((8, 3 * E), lambda b, t: (0, 0)),
        ],
        out_specs=pl.BlockSpec((1, TB, 3 * E), lambda b, t: (b, t, 0)),
        out_shape=jax.ShapeDtypeStruct((B, T, 3 * E), jnp.float32),
    )(hidden_states, w_all, bias_all)

    fc, fs, ict, ist = (jnp.asarray(a) for a in _dft_consts())
    ne = E // EB
    sr3, si3 = pl.pallas_call(
        _corr_body,
        grid=(B, ne),
        in_specs=[
            pl.BlockSpec((1, T, EB), lambda b, e: (b, 0, e)),
            pl.BlockSpec((1, T, EB), lambda b, e: (b, 0, ne + e)),
            pl.BlockSpec((NFP, T), lambda b, e: (0, 0)),
            pl.BlockSpec((NFP, T), lambda b, e: (0, 0)),
        ],
        out_specs=[
            pl.BlockSpec((1, NFP, 1), lambda b, e: (b, 0, 0)),
            pl.BlockSpec((1, NFP, 1), lambda b, e: (b, 0, 0)),
        ],
        out_shape=[
            jax.ShapeDtypeStruct((B, NFP, 1), jnp.float32),
            jax.ShapeDtypeStruct((B, NFP, 1), jnp.float32),
        ],
    )(qkv, qkv, fc, fs)

    delays, weights = pl.pallas_call(
        _invtopk_body,
        in_specs=[
            pl.BlockSpec((B, NFP), lambda: (0, 0)),
            pl.BlockSpec((B, NFP), lambda: (0, 0)),
            pl.BlockSpec((NFP, T), lambda: (0, 0)),
            pl.BlockSpec((NFP, T), lambda: (0, 0)),
        ],
        out_specs=[
            pl.BlockSpec((B, 128), lambda: (0, 0)),
            pl.BlockSpec((B, 128), lambda: (0, 0)),
        ],
        out_shape=[
            jax.ShapeDtypeStruct((B, 128), jnp.int32),
            jax.ShapeDtypeStruct((B, 128), jnp.float32),
        ],
    )(sr3.reshape(B, NFP), si3.reshape(B, NFP), ict, ist)

    ob = jnp.tile(o_b[None, :], (8, 1))
    out = pl.pallas_call(
        _agg_body,
        grid_spec=pltpu.PrefetchScalarGridSpec(
            num_scalar_prefetch=2,
            grid=(B, NG),
            in_specs=[
                pl.BlockSpec((1, T, GW),
                             lambda b, c, dd, ww: (b, 0, 2 * E // GW + c)),
                pl.BlockSpec((E, GW), lambda b, c, dd, ww: (0, c)),
                pl.BlockSpec((8, E), lambda b, c, dd, ww: (0, 0)),
            ],
            out_specs=pl.BlockSpec((1, T, E), lambda b, c, dd, ww: (b, 0, 0)),
            scratch_shapes=[pltpu.VMEM((2 * T, 128), jnp.float32),
                            pltpu.VMEM((2 * T, 128), jnp.float32)],
        ),
        out_shape=jax.ShapeDtypeStruct((B, T, E), jnp.float32),
    )(delays, weights, qkv, o_w_p, ob)
    return out


# speedup vs baseline: 1.2100x; 1.0230x over previous
import functools
import math

import jax
import jax.numpy as jnp
import numpy as np
from jax.experimental import pallas as pl
from jax.experimental.pallas import tpu as pltpu

B, T, E, H = 4, 2048, 1024, 16
DH = E // H
TOPK = int(3 * math.log(T))
NF = T // 2 + 1
NFP = 1152
FB = 384
EB = 512
TB = 512
NG = B
GW = H // NG * DH

_PERM = [h for c in range(NG) for h in range(c, H, NG)]


@functools.lru_cache(maxsize=1)
def _dft_consts():
    f = np.arange(NFP, dtype=np.int64)[:, None]
    t = np.arange(T, dtype=np.int64)[None, :]
    ang = 2.0 * np.pi * ((f * t) % T).astype(np.float64) / T
    valid = (f < NF).astype(np.float64)
    fc = (np.cos(ang) * valid).astype(np.float32)
    fs = (-np.sin(ang) * valid).astype(np.float32)
    wgt = np.where((f == 0) | (f == T // 2), 1.0, 2.0) * valid / (T * E)
    ict = (np.cos(ang) * wgt).astype(np.float32)
    ist = (-np.sin(ang) * wgt).astype(np.float32)
    return fc, fs, ict, ist


def _qkv_body(hs_ref, w_ref, bias_ref, out_ref):
    acc = jax.lax.dot_general(
        hs_ref[0], w_ref[...],
        (((1,), (1,)), ((), ())),
        preferred_element_type=jnp.float32)
    out_ref[0] = acc + bias_ref[0:1, :]


def _corr_body(q_ref, k_ref, fc_ref, fs_ref, sr_ref, si_ref):
    e = pl.program_id(1)
    q = q_ref[0]
    k = k_ref[0]
    dot = functools.partial(jnp.dot, preferred_element_type=jnp.float32)
    qc = dot(fc_ref[...], q)
    qs = dot(fs_ref[...], q)
    kc = dot(fc_ref[...], k)
    ks = dot(fs_ref[...], k)
    sr = jnp.sum(qc * kc + qs * ks, axis=1, keepdims=True)
    si = jnp.sum(qs * kc - qc * ks, axis=1, keepdims=True)

    @pl.when(e == 0)
    def _():
        sr_ref[0] = jnp.zeros_like(sr_ref[0])
        si_ref[0] = jnp.zeros_like(si_ref[0])

    sr_ref[0] += sr
    si_ref[0] += si


def _invtopk_body(srt_ref, sit_ref, ict_ref, ist_ref, d_ref, w_ref):
    dot = functools.partial(jnp.dot, preferred_element_type=jnp.float32)
    vals = dot(srt_ref[...], ict_ref[...]) + dot(sit_ref[...], ist_ref[...])
    lane = jax.lax.broadcasted_iota(jnp.int32, (B, T), 1)
    out_lane = jax.lax.broadcasted_iota(jnp.int32, (B, 128), 1)
    dacc = jnp.zeros((B, 128), jnp.int32)
    wacc = jnp.zeros((B, 128), jnp.float32)
    for i in range(TOPK):
        cur = jnp.max(vals, axis=1, keepdims=True)
        idx = jnp.min(jnp.where(vals == cur, lane, T),
                      axis=1, keepdims=True)
        dacc = jnp.where(out_lane == i, idx, dacc)
        wacc = jnp.where(out_lane == i, cur, wacc)
        vals = jnp.where(lane == idx, -jnp.inf, vals)
    d_ref[...] = dacc
    w_ref[...] = wacc


def _agg_body(d_ref, w_ref, v_ref, ow_ref, ob_ref, out_ref, vd0, vd1):
    c = pl.program_id(1)
    vd0[0:T] = v_ref[0, :, 0:128]
    vd0[T:2 * T] = v_ref[0, :, 0:128]
    vd1[0:T] = v_ref[0, :, 128:256]
    vd1[T:2 * T] = v_ref[0, :, 128:256]
    acc0 = jnp.zeros((T, 128), jnp.float32)
    acc1 = jnp.zeros((T, 128), jnp.float32)
    for i in range(TOPK):
        d = d_ref[c, i]
        w = w_ref[c, i]
        acc0 = acc0 + w * vd0[pl.ds(d, T), :]
        acc1 = acc1 + w * vd1[pl.ds(d, T), :]
    acc = jnp.concatenate([acc0, acc1], axis=1)
    ctr = jax.lax.dot_general(
        acc, ow_ref[...],
        (((1,), (1,)), ((), ())),
        preferred_element_type=jnp.float32)

    @pl.when(c == 0)
    def _():
        out_ref[0] = jnp.broadcast_to(ob_ref[0:1, :], (T, E))

    out_ref[0] += ctr


def kernel(hidden_states, q_w, q_b, k_w, k_b, v_w, v_b, o_w, o_b):
    perm = jnp.asarray(_PERM, dtype=jnp.int32)
    v_w_p = v_w.reshape(H, DH, E)[perm].reshape(E, E)
    v_b_p = v_b.reshape(H, DH)[perm].reshape(E)
    o_w_p = o_w.reshape(E, H, DH)[:, perm].reshape(E, E)
    w_all = jnp.concatenate([q_w, k_w, v_w_p], axis=0)
    bias_all = jnp.tile(
        jnp.concatenate([q_b, k_b, v_b_p])[None, :], (8, 1))

    qkv = pl.pallas_call(
        _qkv_body,
        grid=(B, T // TB),
        in_specs=[
            pl.BlockSpec((1, TB, E), lambda b, t: (b, t, 0)),
            pl.BlockSpec((3 * E, E), lambda b, t: (0, 0)),
            pl.BlockSpec((8, 3 * E), lambda b, t: (0, 0)),
        ],
        out_specs=pl.BlockSpec((1, TB, 3 * E), lambda b, t: (b, t, 0)),
        out_shape=jax.ShapeDtypeStruct((B, T, 3 * E), jnp.float32),
    )(hidden_states, w_all, bias_all)

    fc, fs, ict, ist = (jnp.asarray(a) for a in _dft_consts())
    ne = E // EB
    sr3, si3 = pl.pallas_call(
        _corr_body,
        grid=(B, ne),
        in_specs=[
            pl.BlockSpec((1, T, EB), lambda b, e: (b, 0, e)),
            pl.BlockSpec((1, T, EB), lambda b, e: (b, 0, ne + e)),
            pl.BlockSpec((NFP, T), lambda b, e: (0, 0)),
            pl.BlockSpec((NFP, T), lambda b, e: (0, 0)),
        ],
        out_specs=[
            pl.BlockSpec((1, NFP, 1), lambda b, e: (b, 0, 0)),
            pl.BlockSpec((1, NFP, 1), lambda b, e: (b, 0, 0)),
        ],
        out_shape=[
            jax.ShapeDtypeStruct((B, NFP, 1), jnp.float32),
            jax.ShapeDtypeStruct((B, NFP, 1), jnp.float32),
        ],
    )(qkv, qkv, fc, fs)

    delays, weights = pl.pallas_call(
        _invtopk_body,
        in_specs=[
            pl.BlockSpec((B, NFP), lambda: (0, 0)),
            pl.BlockSpec((B, NFP), lambda: (0, 0)),
            pl.BlockSpec((NFP, T), lambda: (0, 0)),
            pl.BlockSpec((NFP, T), lambda: (0, 0)),
        ],
        out_specs=[
            pl.BlockSpec((B, 128), lambda: (0, 0)),
            pl.BlockSpec((B, 128), lambda: (0, 0)),
        ],
        out_shape=[
            jax.ShapeDtypeStruct((B, 128), jnp.int32),
            jax.ShapeDtypeStruct((B, 128), jnp.float32),
        ],
    )(sr3.reshape(B, NFP), si3.reshape(B, NFP), ict, ist)

    ob = jnp.tile(o_b[None, :], (8, 1))
    out = pl.pallas_call(
        _agg_body,
        grid_spec=pltpu.PrefetchScalarGridSpec(
            num_scalar_prefetch=2,
            grid=(B, NG),
            in_specs=[
                pl.BlockSpec((1, T, GW),
                             lambda b, c, dd, ww: (b, 0, 2 * E // GW + c)),
                pl.BlockSpec((E, GW), lambda b, c, dd, ww: (0, c)),
                pl.BlockSpec((8, E), lambda b, c, dd, ww: (0, 0)),
            ],
            out_specs=pl.BlockSpec((1, T, E), lambda b, c, dd, ww: (b, 0, 0)),
            scratch_shapes=[pltpu.VMEM((2 * T, 128), jnp.float32),
                            pltpu.VMEM((2 * T, 128), jnp.float32)],
        ),
        out_shape=jax.ShapeDtypeStruct((B, T, E), jnp.float32),
    )(delays, weights, qkv, o_w_p, ob)
    return out
